# baseline (speedup 1.0000x reference)
import functools

import jax
import jax.numpy as jnp
from jax import lax
from jax.experimental import pallas as pl
from jax.experimental.pallas import tpu as pltpu
from jax.experimental.pallas import tpu_sc as plsc

_B, _Q, _C, _L = 16, 20000, 91, 20
_CP = 128
_LP = 32
_NC = 2


def _tc_reduce_body(x_ref, vals_ref, idx_ref):
    p = jax.nn.sigmoid(x_ref[0])
    bm = jnp.max(p, axis=1, keepdims=True)
    qio = lax.broadcasted_iota(jnp.int32, (_B, _Q), 1)
    bidx = jnp.min(jnp.where(p == bm, qio, _Q), axis=1, keepdims=True)
    vals_ref[0] = bm
    idx_ref[0] = bidx


def _tc_reduce(logits_t):
    return pl.pallas_call(
        _tc_reduce_body,
        grid=(_C,),
        in_specs=[pl.BlockSpec((1, _B, _Q), lambda c: (c, 0, 0))],
        out_specs=[
            pl.BlockSpec((1, _B, 1), lambda c: (c, 0, 0)),
            pl.BlockSpec((1, _B, 1), lambda c: (c, 0, 0)),
        ],
        out_shape=[
            jax.ShapeDtypeStruct((_C, _B, 1), jnp.float32),
            jax.ShapeDtypeStruct((_C, _B, 1), jnp.int32),
        ],
    )(logits_t)


def _sc_gather_body(vals_hbm, idx_hbm, lab_hbm, boxes_hbm,
                    scores_out, boxes_out,
                    lab_v, gidx_v, sc_v, bidx_v, brow_v, sem):
    wid = lax.axis_index("s") * _NC + lax.axis_index("c")
    b = wid // 2
    pltpu.sync_copy(lab_hbm.at[wid], lab_v)
    gidx_v[...] = lab_v[...] + b * _CP
    pltpu.async_copy(vals_hbm.at[gidx_v], sc_v, sem).wait()
    pltpu.sync_copy(sc_v, scores_out.at[wid])
    pltpu.async_copy(idx_hbm.at[gidx_v], bidx_v, sem).wait()
    bidx_v[...] = bidx_v[...] + b * (4 * _Q)
    for k in range(4):
        gidx_v[...] = bidx_v[...] + k * _Q
        pltpu.async_copy(boxes_hbm.at[gidx_v], brow_v.at[k], sem).wait()
    pltpu.sync_copy(brow_v, boxes_out.at[wid])


@functools.cache
def _sc_gather():
    return functools.partial(
        pl.kernel,
        mesh=plsc.VectorSubcoreMesh(core_axis_name="c", subcore_axis_name="s"),
        compiler_params=pltpu.CompilerParams(use_tc_tiling_on_sc=False),
        out_type=[
            jax.ShapeDtypeStruct((_B * 2, 16), jnp.float32),
            jax.ShapeDtypeStruct((_B * 2, 4, 16), jnp.float32),
        ],
        scratch_types=[
            pltpu.VMEM((16,), jnp.int32),
            pltpu.VMEM((16,), jnp.int32),
            pltpu.VMEM((16,), jnp.float32),
            pltpu.VMEM((16,), jnp.int32),
            pltpu.VMEM((4, 16), jnp.float32),
            pltpu.SemaphoreType.DMA,
        ],
    )(_sc_gather_body)


def kernel(pred_logits, pred_boxes, target_sizes, target_labels):
    del target_sizes
    logits_t = pred_logits.transpose(2, 0, 1)
    vals_cb, idx_cb = _tc_reduce(logits_t)
    vals = jnp.pad(vals_cb[:, :, 0].T, ((0, 0), (0, _CP - _C))).reshape(-1)
    idx = jnp.pad(idx_cb[:, :, 0].T, ((0, 0), (0, _CP - _C))).reshape(-1)
    lab = jnp.pad(target_labels, ((0, 0), (0, _LP - _L))).reshape(_B * 2, 16)
    boxes_kq = pred_boxes.transpose(0, 2, 1).reshape(-1)
    scores32, boxes32 = _sc_gather()(vals, idx, lab, boxes_kq)
    scores = scores32.reshape(_B, _LP)[:, :_L]
    boxes = (boxes32.reshape(_B, 2, 4, 16).transpose(0, 1, 3, 2)
             .reshape(_B, _LP, 4)[:, :_L, :])
    return (scores, target_labels, boxes)
